# baseline (speedup 1.0000x reference)
import jax
import jax.numpy as jnp
from jax import lax
from jax.experimental import pallas as pl
from jax.experimental.pallas import tpu as pltpu
from jax.experimental.pallas import tpu_sc as plsc

VOCAB = 1000000
HIDDEN = 64
B = 4096
L = 200

NC, NS = 2, 16
NW = NC * NS
BPW = B // NW
RPC = 4
NITER = BPW // RPC
LP = 256
SPLITS = ((0, 128), (128, 72))


def _prep_body(enc_ref, dow_ref, hod_ref, dow_emb_ref, hod_emb_ref,
               encp_ref, comb_ref, fused_ref):
    encp_ref[...] = jnp.pad(enc_ref[...], ((0, 0), (0, LP - L)))
    comb_ref[...] = jnp.pad(dow_ref[...] * 25 + hod_ref[...],
                            ((0, 0), (0, LP - L)))
    for d in range(8):
        fused_ref[d * 25:(d + 1) * 25, :] = dow_emb_ref[d:d + 1, :] + hod_emb_ref[...]


def _prep(enc, dow, hod, dow_emb, hod_emb):
    return pl.pallas_call(
        _prep_body,
        out_shape=[
            jax.ShapeDtypeStruct((B, LP), jnp.int32),
            jax.ShapeDtypeStruct((B, LP), jnp.int32),
            jax.ShapeDtypeStruct((200, HIDDEN), jnp.float32),
        ],
    )(enc, dow, hod, dow_emb, hod_emb)


def _sc_body(enc_hbm, comb_hbm, emb_hbm, fused_hbm, out_hbm,
             enc0, enc1, comb0, comb1, rows0, rows1,
             sem_i, sem_g0, sem_g1, sem_f0, sem_f1, sem_o0, sem_o1):
    wid = lax.axis_index("s") * NC + lax.axis_index("c")
    b0 = wid * BPW

    encb = (enc0, enc1)
    combb = (comb0, comb1)
    rows = (rows0, rows1)
    sem_g = (sem_g0, sem_g1)
    sem_f = (sem_f0, sem_f1)
    sem_o = (sem_o0, sem_o1)

    def stage_idx(t, p):
        bb = b0 + t * RPC
        pltpu.async_copy(enc_hbm.at[pl.ds(bb, RPC)], encb[p], sem_i)
        pltpu.async_copy(comb_hbm.at[pl.ds(bb, RPC)], combb[p], sem_i)

    def wait_idx(p):
        pltpu.make_async_copy(enc_hbm.at[pl.ds(0, RPC)], encb[p], sem_i).wait()
        pltpu.make_async_copy(comb_hbm.at[pl.ds(0, RPC)], combb[p], sem_i).wait()

    def fire_main(p):
        for i in range(RPC):
            for off, n in SPLITS:
                sl = pl.ds(off, n)
                pltpu.async_copy(
                    emb_hbm.at[encb[p].at[i, sl]], rows[p].at[i, sl], sem_g[p])

    def wait_main(p):
        for i in range(RPC):
            for off, n in SPLITS:
                sl = pl.ds(off, n)
                pltpu.make_async_copy(
                    emb_hbm.at[encb[p].at[i, sl]], rows[p].at[i, sl],
                    sem_g[p]).wait()

    def fire_fused(p):
        for i in range(RPC):
            for off, n in SPLITS:
                sl = pl.ds(off, n)
                pltpu.async_copy(
                    fused_hbm.at[combb[p].at[i, sl]], rows[p].at[i, sl],
                    sem_f[p], add=True)

    def wait_fused(p):
        for i in range(RPC):
            for off, n in SPLITS:
                sl = pl.ds(off, n)
                pltpu.make_async_copy(
                    fused_hbm.at[combb[p].at[i, sl]], rows[p].at[i, sl],
                    sem_f[p]).wait()

    def fire_out(t, p):
        bb = b0 + t * RPC
        pltpu.async_copy(rows[p], out_hbm.at[pl.ds(bb, RPC)], sem_o[p])

    def drain_out(p):
        pltpu.make_async_copy(rows[p], out_hbm.at[pl.ds(b0, RPC)],
                              sem_o[p]).wait()

    stage_idx(0, 0)
    stage_idx(1, 1)
    wait_idx(0)
    fire_main(0)

    def full_step(t, p, drain):
        wait_main(p)
        fire_fused(p)
        wait_idx(1 - p)
        if drain:
            drain_out(1 - p)
        fire_main(1 - p)
        wait_fused(p)
        stage_idx(t + 2, p)
        fire_out(t, p)

    full_step(0, 0, False)
    full_step(1, 1, True)

    def pair(g, c):
        t = 2 * g
        full_step(t, 0, True)
        full_step(t + 1, 1, True)
        return c

    lax.fori_loop(1, NITER // 2 - 1, pair, 0)

    t = NITER - 2
    wait_main(0)
    fire_fused(0)
    wait_idx(1)
    drain_out(1)
    fire_main(1)
    wait_fused(0)
    fire_out(t, 0)

    wait_main(1)
    fire_fused(1)
    wait_fused(1)
    fire_out(t + 1, 1)

    drain_out(0)
    drain_out(1)


def _sc_lookup(enc, comb, embedding, fused):
    mesh = plsc.VectorSubcoreMesh(core_axis_name="c", subcore_axis_name="s")
    k = pl.kernel(
        _sc_body,
        out_type=jax.ShapeDtypeStruct((B, L, HIDDEN), jnp.float32),
        mesh=mesh,
        compiler_params=pltpu.CompilerParams(use_tc_tiling_on_sc=False),
        scratch_types=[
            pltpu.VMEM((RPC, LP), jnp.int32),
            pltpu.VMEM((RPC, LP), jnp.int32),
            pltpu.VMEM((RPC, LP), jnp.int32),
            pltpu.VMEM((RPC, LP), jnp.int32),
            pltpu.VMEM((RPC, L, HIDDEN), jnp.float32),
            pltpu.VMEM((RPC, L, HIDDEN), jnp.float32),
            pltpu.SemaphoreType.DMA,
            pltpu.SemaphoreType.DMA,
            pltpu.SemaphoreType.DMA,
            pltpu.SemaphoreType.DMA,
            pltpu.SemaphoreType.DMA,
            pltpu.SemaphoreType.DMA,
            pltpu.SemaphoreType.DMA,
        ],
    )
    return k(enc, comb, embedding, fused)


def kernel(enc_src, DoW, HoD, embedding, DoW_Emb, HoD_Emb):
    enc = jnp.asarray(enc_src, jnp.int32)
    dow = jnp.asarray(DoW, jnp.int32)
    hod = jnp.asarray(HoD, jnp.int32)
    encp, comb, fused = _prep(enc, dow, hod, DoW_Emb.astype(jnp.float32),
                              HoD_Emb.astype(jnp.float32))
    return _sc_lookup(encp, comb, embedding.astype(jnp.float32), fused)


# speedup vs baseline: 1.0027x; 1.0027x over previous
import jax
import jax.numpy as jnp
from jax import lax
from jax.experimental import pallas as pl
from jax.experimental.pallas import tpu as pltpu
from jax.experimental.pallas import tpu_sc as plsc

VOCAB = 1000000
HIDDEN = 64
B = 4096
L = 200

NC, NS = 2, 16
NW = NC * NS
BPW = B // NW
RPC = 4
NITER = BPW // RPC
LP = 256
SPLITS = ((0, 128), (128, 72))


def _prep_body(enc_ref, dow_ref, hod_ref, dow_emb_ref, hod_emb_ref,
               encp_ref, comb_ref, fused_ref):
    encp_ref[...] = jnp.pad(enc_ref[...], ((0, 0), (0, LP - L)))
    comb_ref[...] = jnp.pad(dow_ref[...] * 25 + hod_ref[...],
                            ((0, 0), (0, LP - L)))
    for d in range(8):
        fused_ref[d * 25:(d + 1) * 25, :] = dow_emb_ref[d:d + 1, :] + hod_emb_ref[...]


def _prep(enc, dow, hod, dow_emb, hod_emb):
    return pl.pallas_call(
        _prep_body,
        out_shape=[
            jax.ShapeDtypeStruct((B, LP), jnp.int32),
            jax.ShapeDtypeStruct((B, LP), jnp.int32),
            jax.ShapeDtypeStruct((200, HIDDEN), jnp.float32),
        ],
    )(enc, dow, hod, dow_emb, hod_emb)


def _sc_body(enc_hbm, comb_hbm, emb_hbm, fused_hbm, out_hbm,
             enc0, enc1, comb0, comb1, rows0, rows1,
             sem_i, sem_g0, sem_g1, sem_f0, sem_f1, sem_o0, sem_o1):
    wid = lax.axis_index("s") * NC + lax.axis_index("c")
    b0 = wid * BPW
    f0 = b0 * L

    encb = (enc0, enc1)
    combb = (comb0, comb1)
    rows = (rows0, rows1)
    sem_g = (sem_g0, sem_g1)
    sem_f = (sem_f0, sem_f1)
    sem_o = (sem_o0, sem_o1)

    def stage_idx(t, p):
        bb = b0 + t * RPC
        pltpu.async_copy(enc_hbm.at[pl.ds(bb, RPC)], encb[p], sem_i)
        pltpu.async_copy(comb_hbm.at[pl.ds(bb, RPC)], combb[p], sem_i)

    def wait_idx(p):
        pltpu.make_async_copy(enc_hbm.at[pl.ds(0, RPC)], encb[p], sem_i).wait()
        pltpu.make_async_copy(comb_hbm.at[pl.ds(0, RPC)], combb[p], sem_i).wait()

    def fire_main(p):
        for i in range(RPC):
            for off, n in SPLITS:
                sl = pl.ds(off, n)
                dst = pl.ds(i * L + off, n)
                pltpu.async_copy(
                    emb_hbm.at[encb[p].at[i, sl]], rows[p].at[dst], sem_g[p])

    def wait_main(p):
        for i in range(RPC):
            for off, n in SPLITS:
                sl = pl.ds(off, n)
                dst = pl.ds(i * L + off, n)
                pltpu.make_async_copy(
                    emb_hbm.at[encb[p].at[i, sl]], rows[p].at[dst],
                    sem_g[p]).wait()

    def fire_fused(p):
        for i in range(RPC):
            for off, n in SPLITS:
                sl = pl.ds(off, n)
                dst = pl.ds(i * L + off, n)
                pltpu.async_copy(
                    fused_hbm.at[combb[p].at[i, sl]], rows[p].at[dst],
                    sem_f[p], add=True)

    def wait_fused(p):
        for i in range(RPC):
            for off, n in SPLITS:
                sl = pl.ds(off, n)
                dst = pl.ds(i * L + off, n)
                pltpu.make_async_copy(
                    fused_hbm.at[combb[p].at[i, sl]], rows[p].at[dst],
                    sem_f[p]).wait()

    def fire_out(t, p):
        fb = f0 + t * RPC * L
        pltpu.async_copy(rows[p], out_hbm.at[pl.ds(fb, RPC * L)], sem_o[p])

    def drain_out(p):
        pltpu.make_async_copy(rows[p], out_hbm.at[pl.ds(f0, RPC * L)],
                              sem_o[p]).wait()

    stage_idx(0, 0)
    stage_idx(1, 1)
    wait_idx(0)
    fire_main(0)

    def full_step(t, p, drain):
        wait_main(p)
        fire_fused(p)
        wait_idx(1 - p)
        if drain:
            drain_out(1 - p)
        fire_main(1 - p)
        wait_fused(p)
        stage_idx(t + 2, p)
        fire_out(t, p)

    full_step(0, 0, False)
    full_step(1, 1, True)

    def pair(g, c):
        t = 2 * g
        full_step(t, 0, True)
        full_step(t + 1, 1, True)
        return c

    lax.fori_loop(1, NITER // 2 - 1, pair, 0)

    t = NITER - 2
    wait_main(0)
    fire_fused(0)
    wait_idx(1)
    drain_out(1)
    fire_main(1)
    wait_fused(0)
    fire_out(t, 0)

    wait_main(1)
    fire_fused(1)
    wait_fused(1)
    fire_out(t + 1, 1)

    drain_out(0)
    drain_out(1)


def _sc_lookup(enc, comb, embedding, fused):
    mesh = plsc.VectorSubcoreMesh(core_axis_name="c", subcore_axis_name="s")
    k = pl.kernel(
        _sc_body,
        out_type=jax.ShapeDtypeStruct((B * L, HIDDEN), jnp.float32),
        mesh=mesh,
        compiler_params=pltpu.CompilerParams(use_tc_tiling_on_sc=False),
        scratch_types=[
            pltpu.VMEM((RPC, LP), jnp.int32),
            pltpu.VMEM((RPC, LP), jnp.int32),
            pltpu.VMEM((RPC, LP), jnp.int32),
            pltpu.VMEM((RPC, LP), jnp.int32),
            pltpu.VMEM((RPC * L, HIDDEN), jnp.float32),
            pltpu.VMEM((RPC * L, HIDDEN), jnp.float32),
            pltpu.SemaphoreType.DMA,
            pltpu.SemaphoreType.DMA,
            pltpu.SemaphoreType.DMA,
            pltpu.SemaphoreType.DMA,
            pltpu.SemaphoreType.DMA,
            pltpu.SemaphoreType.DMA,
            pltpu.SemaphoreType.DMA,
        ],
    )
    return k(enc, comb, embedding, fused)


def kernel(enc_src, DoW, HoD, embedding, DoW_Emb, HoD_Emb):
    enc = jnp.asarray(enc_src, jnp.int32)
    dow = jnp.asarray(DoW, jnp.int32)
    hod = jnp.asarray(HoD, jnp.int32)
    encp, comb, fused = _prep(enc, dow, hod, DoW_Emb.astype(jnp.float32),
                              HoD_Emb.astype(jnp.float32))
    out = _sc_lookup(encp, comb, embedding.astype(jnp.float32), fused)
    return out.reshape(B, L, HIDDEN)


# speedup vs baseline: 1.0886x; 1.0857x over previous
import functools

import jax
import jax.numpy as jnp
from jax import lax
from jax.experimental import pallas as pl
from jax.experimental.pallas import tpu as pltpu
from jax.experimental.pallas import tpu_sc as plsc

VOCAB = 1000000
HIDDEN = 64
B = 4096
L = 200
N = B * L

NC, NS, LANES = 2, 16, 16
NW = NC * NS
IDX_W = 128
SUB = 8
CHUNK = SUB * IDX_W
PER_W = N // NW
ROWS_PER_W = PER_W // IDX_W
NITER = PER_W // CHUNK


def _fuse_body(dow_ref, hod_ref, out_ref):
    for d in range(8):
        out_ref[d * 25:(d + 1) * 25, :] = dow_ref[d:d + 1, :] + hod_ref[...]


def _fuse_tables(dow_emb, hod_emb):
    return pl.pallas_call(
        _fuse_body,
        out_shape=jax.ShapeDtypeStruct((200, HIDDEN), jnp.float32),
    )(dow_emb, hod_emb)


def _sc_body(enc_hbm, dow_hbm, hod_hbm, emb_hbm, fused_hbm, out_hbm,
             idx_a, idx_b, dv, rows_a, sem, semi):
    wid = lax.axis_index("s") * NC + lax.axis_index("c")
    row0 = wid * ROWS_PER_W

    def chunk_body(t, carry):
        rbase = row0 + t * SUB
        obase = (row0 + t * SUB) * IDX_W

        c1 = pltpu.async_copy(enc_hbm.at[pl.ds(rbase, SUB)], idx_a, semi)
        c2 = pltpu.async_copy(dow_hbm.at[pl.ds(rbase, SUB)], dv, semi)
        c3 = pltpu.async_copy(hod_hbm.at[pl.ds(rbase, SUB)], idx_b, semi)
        c1.wait(); c2.wait(); c3.wait()

        for j in range(SUB):
            for i in range(IDX_W // LANES):
                sl = pl.ds(i * LANES, LANES)
                idx_b[j, sl] = dv[j, sl] * 25 + idx_b[j, sl]

        copies = []
        for j in range(SUB):
            dst = pl.ds(j * IDX_W, IDX_W)
            copies.append(pltpu.async_copy(
                emb_hbm.at[idx_a.at[j]], rows_a.at[dst], sem))
        for cp in copies:
            cp.wait()

        copies = []
        for j in range(SUB):
            dst = pl.ds(j * IDX_W, IDX_W)
            copies.append(pltpu.async_copy(
                fused_hbm.at[idx_b.at[j]], rows_a.at[dst], sem, add=True))
        for cp in copies:
            cp.wait()

        pltpu.sync_copy(rows_a, out_hbm.at[pl.ds(obase, CHUNK)])
        return carry

    lax.fori_loop(0, NITER, chunk_body, 0)


def _sc_lookup(enc2, dow2, hod2, embedding, fused):
    mesh = plsc.VectorSubcoreMesh(core_axis_name="c", subcore_axis_name="s")
    k = pl.kernel(
        _sc_body,
        out_type=jax.ShapeDtypeStruct((N, HIDDEN), jnp.float32),
        mesh=mesh,
        compiler_params=pltpu.CompilerParams(use_tc_tiling_on_sc=False),
        scratch_types=[
            pltpu.VMEM((SUB, IDX_W), jnp.int32),
            pltpu.VMEM((SUB, IDX_W), jnp.int32),
            pltpu.VMEM((SUB, IDX_W), jnp.int32),
            pltpu.VMEM((CHUNK, HIDDEN), jnp.float32),
            pltpu.SemaphoreType.DMA,
            pltpu.SemaphoreType.DMA,
        ],
    )
    return k(enc2, dow2, hod2, embedding, fused)


def kernel(enc_src, DoW, HoD, embedding, DoW_Emb, HoD_Emb):
    enc2 = jnp.asarray(enc_src, jnp.int32).reshape(N // IDX_W, IDX_W)
    dow2 = jnp.asarray(DoW, jnp.int32).reshape(N // IDX_W, IDX_W)
    hod2 = jnp.asarray(HoD, jnp.int32).reshape(N // IDX_W, IDX_W)
    fused = _fuse_tables(DoW_Emb.astype(jnp.float32), HoD_Emb.astype(jnp.float32))
    out = _sc_lookup(enc2, dow2, hod2, embedding.astype(jnp.float32), fused)
    return out.reshape(B, L, HIDDEN)
